# baseline (speedup 1.0000x reference)
import functools

import jax
import jax.numpy as jnp
from jax import lax
from jax.experimental import pallas as pl
from jax.experimental.pallas import tpu as pltpu
from jax.experimental.pallas import tpu_sc as plsc

N = 10000
NP = 10240
E = 320000
EP = 327680
D_IN = 128
D_HID = 128
NCLS = 40
NC = 2
NS = 16
CHUNK = 128
EW = EP // (NC * NS)
RW = EW // CHUNK
RPT = NP // NS
RB = 5000


def _sc_mesh():
  return plsc.VectorSubcoreMesh(core_axis_name="c", subcore_axis_name="s",
                                num_cores=NC, num_subcores=NS)


def _scatter_l0(table, srcm, dstm, zfeat, zdeg):
  def body(table, srcm, dstm, zfeat, zdeg, acc_out, deg_out,
           sw, dst_v, rows0, rows1, ones_v, acc_sh, deg_sh,
           g0, g1, is0, is1, zsem):
    c = lax.axis_index("c")
    s = lax.axis_index("s")
    w = c * NS + s
    r0 = s * RPT
    z0 = pltpu.async_copy(zfeat.at[pl.ds(r0, RPT)], acc_sh.at[pl.ds(r0, RPT)], zsem)
    z1 = pltpu.async_copy(zdeg.at[pl.ds(r0, RPT)], deg_sh.at[pl.ds(r0, RPT)], zsem)
    rb = w * RW
    pltpu.sync_copy(dstm.at[pl.ds(rb, RW)], dst_v)
    for i in range(CHUNK // 16):
      ones_v[pl.ds(i * 16, 16)] = jnp.full((16,), 1.0, jnp.float32)
    pltpu.sync_copy(srcm.at[pl.ds(rb, 1)], sw.at[0])
    pltpu.sync_copy(srcm.at[pl.ds(rb + 1, 1)], sw.at[1])
    pltpu.async_copy(table.at[sw.at[0, 0]], rows0, g0)
    pltpu.async_copy(table.at[sw.at[1, 0]], rows1, g1)
    z0.wait()
    z1.wait()
    plsc.subcore_barrier()

    def step(i, carry):
      k = 2 * i
      pltpu.make_async_copy(table.at[sw.at[0, 0]], rows0, g0).wait()
      pltpu.async_copy(srcm.at[pl.ds(rb + k + 2, 1)], sw.at[0], is0)
      pltpu.sync_copy(ones_v, deg_sh.at[dst_v.at[k]], add=True)
      pltpu.sync_copy(rows0, acc_sh.at[dst_v.at[k]], add=True)
      pltpu.make_async_copy(srcm.at[pl.ds(rb + k + 2, 1)], sw.at[0], is0).wait()
      pltpu.async_copy(table.at[sw.at[0, 0]], rows0, g0)
      pltpu.make_async_copy(table.at[sw.at[1, 0]], rows1, g1).wait()
      pltpu.async_copy(srcm.at[pl.ds(rb + k + 3, 1)], sw.at[1], is1)
      pltpu.sync_copy(ones_v, deg_sh.at[dst_v.at[k + 1]], add=True)
      pltpu.sync_copy(rows1, acc_sh.at[dst_v.at[k + 1]], add=True)
      pltpu.make_async_copy(srcm.at[pl.ds(rb + k + 3, 1)], sw.at[1], is1).wait()
      pltpu.async_copy(table.at[sw.at[1, 0]], rows1, g1)
      return carry

    lax.fori_loop(0, RW // 2, step, 0)
    pltpu.make_async_copy(table.at[sw.at[0, 0]], rows0, g0).wait()
    pltpu.make_async_copy(table.at[sw.at[1, 0]], rows1, g1).wait()
    plsc.subcore_barrier()
    o0 = pltpu.async_copy(acc_sh.at[pl.ds(r0, RPT)], acc_out.at[c, pl.ds(r0, RPT)], zsem)
    o1 = pltpu.async_copy(deg_sh.at[pl.ds(r0, RPT)], deg_out.at[c, pl.ds(r0, RPT)], zsem)
    o0.wait()
    o1.wait()

  f = pl.kernel(
      body,
      out_type=(jax.ShapeDtypeStruct((NC, NP, D_IN), jnp.float32),
                jax.ShapeDtypeStruct((NC, NP), jnp.float32)),
      mesh=_sc_mesh(),
      scratch_types=(
          pltpu.VMEM((2, 1, CHUNK), jnp.int32),
          pltpu.VMEM((RW, CHUNK), jnp.int32),
          pltpu.VMEM((CHUNK, D_IN), jnp.float32),
          pltpu.VMEM((CHUNK, D_IN), jnp.float32),
          pltpu.VMEM((CHUNK,), jnp.float32),
          pltpu.VMEM_SHARED((NP, D_IN), jnp.float32),
          pltpu.VMEM_SHARED((NP,), jnp.float32),
          pltpu.SemaphoreType.DMA,
          pltpu.SemaphoreType.DMA,
          pltpu.SemaphoreType.DMA,
          pltpu.SemaphoreType.DMA,
          pltpu.SemaphoreType.DMA,
      ),
      name="sage_scatter_l0",
  )
  return f(table, srcm, dstm, zfeat, zdeg)


def _scatter_l1(table, srcm, dstm, zp):
  def body(table, srcm, dstm, zp, acc_out, sw, dst_v, rows0, rows1,
           acc_sh, g0, g1, is0, is1, zsem):
    c = lax.axis_index("c")
    s = lax.axis_index("s")
    w = c * NS + s
    r0 = s * RPT
    z0 = pltpu.async_copy(zp.at[pl.ds(r0, RPT)], acc_sh.at[pl.ds(r0, RPT)], zsem)
    rb = w * RW
    pltpu.sync_copy(dstm.at[pl.ds(rb, RW)], dst_v)
    pltpu.sync_copy(srcm.at[pl.ds(rb, 1)], sw.at[0])
    pltpu.sync_copy(srcm.at[pl.ds(rb + 1, 1)], sw.at[1])
    pltpu.async_copy(table.at[sw.at[0, 0]], rows0, g0)
    pltpu.async_copy(table.at[sw.at[1, 0]], rows1, g1)
    z0.wait()
    plsc.subcore_barrier()

    def step(i, carry):
      k = 2 * i
      pltpu.make_async_copy(table.at[sw.at[0, 0]], rows0, g0).wait()
      pltpu.async_copy(srcm.at[pl.ds(rb + k + 2, 1)], sw.at[0], is0)
      pltpu.sync_copy(rows0, acc_sh.at[dst_v.at[k]], add=True)
      pltpu.make_async_copy(srcm.at[pl.ds(rb + k + 2, 1)], sw.at[0], is0).wait()
      pltpu.async_copy(table.at[sw.at[0, 0]], rows0, g0)
      pltpu.make_async_copy(table.at[sw.at[1, 0]], rows1, g1).wait()
      pltpu.async_copy(srcm.at[pl.ds(rb + k + 3, 1)], sw.at[1], is1)
      pltpu.sync_copy(rows1, acc_sh.at[dst_v.at[k + 1]], add=True)
      pltpu.make_async_copy(srcm.at[pl.ds(rb + k + 3, 1)], sw.at[1], is1).wait()
      pltpu.async_copy(table.at[sw.at[1, 0]], rows1, g1)
      return carry

    lax.fori_loop(0, RW // 2, step, 0)
    pltpu.make_async_copy(table.at[sw.at[0, 0]], rows0, g0).wait()
    pltpu.make_async_copy(table.at[sw.at[1, 0]], rows1, g1).wait()
    plsc.subcore_barrier()
    pltpu.sync_copy(acc_sh.at[pl.ds(r0, RPT)], acc_out.at[c, pl.ds(r0, RPT)])

  f = pl.kernel(
      body,
      out_type=jax.ShapeDtypeStruct((NC, NP, D_HID), jnp.float32),
      mesh=_sc_mesh(),
      scratch_types=(
          pltpu.VMEM((2, 1, CHUNK), jnp.int32),
          pltpu.VMEM((RW, CHUNK), jnp.int32),
          pltpu.VMEM((CHUNK, D_HID), jnp.float32),
          pltpu.VMEM((CHUNK, D_HID), jnp.float32),
          pltpu.VMEM_SHARED((NP, D_HID), jnp.float32),
          pltpu.SemaphoreType.DMA,
          pltpu.SemaphoreType.DMA,
          pltpu.SemaphoreType.DMA,
          pltpu.SemaphoreType.DMA,
          pltpu.SemaphoreType.DMA,
      ),
      name="sage_scatter_l1",
  )
  return f(table, srcm, dstm, zp)


def _dense_body(x_ref, acc_ref, degt_ref, ws0_ref, wn0_ref, bs0_ref, bn0_ref,
                ws1_ref, bs1_ref, bn1_ref, h_ref, s_ref):
  deg = degt_ref[:, 0:1] + degt_ref[:, 1:2]
  inv = 1.0 / jnp.maximum(deg, 1.0)
  hn = (acc_ref[0] + acc_ref[1]) * inv
  t = (jnp.dot(x_ref[...], ws0_ref[...], preferred_element_type=jnp.float32)
       + jnp.dot(hn, wn0_ref[...], preferred_element_type=jnp.float32)
       + bs0_ref[...] + bn0_ref[...])
  h = jnp.maximum(t, 0.0)
  nrm = jnp.sqrt(jnp.sum(h * h, axis=1, keepdims=True))
  h = h / jnp.maximum(nrm, 1e-12)
  h_ref[...] = h
  s_ref[...] = (jnp.dot(h, ws1_ref[...], preferred_element_type=jnp.float32)
                + bs1_ref[...] + bn1_ref[...])


def _dense(x, acc, degt, ws0, wn0, bs0, bn0, ws1, bs1, bn1):
  grid = (N // RB,)
  return pl.pallas_call(
      _dense_body,
      grid=grid,
      in_specs=[
          pl.BlockSpec((RB, D_IN), lambda i: (i, 0)),
          pl.BlockSpec((NC, RB, D_IN), lambda i: (0, i, 0)),
          pl.BlockSpec((RB, NC), lambda i: (i, 0)),
          pl.BlockSpec((D_IN, D_HID), lambda i: (0, 0)),
          pl.BlockSpec((D_IN, D_HID), lambda i: (0, 0)),
          pl.BlockSpec((1, D_HID), lambda i: (0, 0)),
          pl.BlockSpec((1, D_HID), lambda i: (0, 0)),
          pl.BlockSpec((D_HID, NCLS), lambda i: (0, 0)),
          pl.BlockSpec((1, NCLS), lambda i: (0, 0)),
          pl.BlockSpec((1, NCLS), lambda i: (0, 0)),
      ],
      out_specs=[
          pl.BlockSpec((RB, D_HID), lambda i: (i, 0)),
          pl.BlockSpec((RB, NCLS), lambda i: (i, 0)),
      ],
      out_shape=[
          jax.ShapeDtypeStruct((N, D_HID), jnp.float32),
          jax.ShapeDtypeStruct((N, NCLS), jnp.float32),
      ],
      name="sage_dense",
  )(x, acc, degt, ws0, wn0, bs0, bn0, ws1, bs1, bn1)


def _combine_body(s_ref, acc1_ref, degt_ref, wn1_ref, o_ref):
  deg = degt_ref[:, 0:1] + degt_ref[:, 1:2]
  inv = 1.0 / jnp.maximum(deg, 1.0)
  hn = (acc1_ref[0] + acc1_ref[1]) * inv
  o_ref[...] = s_ref[...] + jnp.dot(hn, wn1_ref[...],
                                    preferred_element_type=jnp.float32)


def _combine(s, acc1, degt, wn1):
  grid = (N // RB,)
  return pl.pallas_call(
      _combine_body,
      grid=grid,
      in_specs=[
          pl.BlockSpec((RB, NCLS), lambda i: (i, 0)),
          pl.BlockSpec((NC, RB, D_HID), lambda i: (0, i, 0)),
          pl.BlockSpec((RB, NC), lambda i: (i, 0)),
          pl.BlockSpec((D_HID, NCLS), lambda i: (0, 0)),
      ],
      out_specs=pl.BlockSpec((RB, NCLS), lambda i: (i, 0)),
      out_shape=jax.ShapeDtypeStruct((N, NCLS), jnp.float32),
      name="sage_combine",
  )(s, acc1, degt, wn1)


def kernel(features, edge_index, W_self0, W_neigh0, b_self0, b_neigh0,
           W_self1, W_neigh1, b_self1, b_neigh1):
  src = edge_index[0]
  dst = edge_index[1]
  pad = EP - E
  ar = jnp.arange(pad, dtype=jnp.int32)
  pad_src = (ar * 97) % N
  pad_dst = N + (ar % (NP - N))
  srcm = jnp.pad(
      jnp.concatenate([src, pad_src]).reshape(EP // CHUNK, CHUNK),
      ((0, 2), (0, 0)))
  dstm = jnp.pad(
      jnp.concatenate([dst, pad_dst]).reshape(EP // CHUNK, CHUNK),
      ((0, 2), (0, 0)))

  zfeat = jnp.zeros((NP, D_IN), jnp.float32)
  zdeg = jnp.zeros((NP,), jnp.float32)

  acc0, deg = _scatter_l0(features, srcm, dstm, zfeat, zdeg)
  degt = deg.T
  h, s = _dense(features, acc0, degt,
                W_self0, W_neigh0,
                b_self0.reshape(1, D_HID), b_neigh0.reshape(1, D_HID),
                W_self1,
                b_self1.reshape(1, NCLS), b_neigh1.reshape(1, NCLS))
  acc1 = _scatter_l1(h, srcm, dstm, zfeat)
  return _combine(s, acc1, degt, W_neigh1)


# speedup vs baseline: 1.0767x; 1.0767x over previous
import functools

import jax
import jax.numpy as jnp
from jax import lax
from jax.experimental import pallas as pl
from jax.experimental.pallas import tpu as pltpu
from jax.experimental.pallas import tpu_sc as plsc

N = 10000
NP = 10240
E = 320000
EP = 327680
D_IN = 128
D_HID = 128
NCLS = 40
NC = 2
NS = 16
CHUNK = 128
EW = EP // (NC * NS)
RW = EW // CHUNK
RPT = NP // NS
RB = 5000


def _sc_mesh():
  return plsc.VectorSubcoreMesh(core_axis_name="c", subcore_axis_name="s",
                                num_cores=NC, num_subcores=NS)


def _scatter_l0(table, srcm, dstm, zfeat, zdeg):
  def body(table, srcm, dstm, zfeat, zdeg, acc_out, deg_out,
           sw, dst_v, rows0, rows1, ones_v, acc_sh, deg_sh,
           g0, g1, is0, is1, zsem):
    c = lax.axis_index("c")
    s = lax.axis_index("s")
    w = c * NS + s
    r0 = s * RPT
    z0 = pltpu.async_copy(zfeat.at[pl.ds(r0, RPT)], acc_sh.at[pl.ds(r0, RPT)], zsem)
    z1 = pltpu.async_copy(zdeg.at[pl.ds(r0, RPT)], deg_sh.at[pl.ds(r0, RPT)], zsem)
    rb = w * RW
    pltpu.sync_copy(dstm.at[pl.ds(rb, RW)], dst_v)
    for i in range(CHUNK // 16):
      ones_v[pl.ds(i * 16, 16)] = jnp.full((16,), 1.0, jnp.float32)
    pltpu.sync_copy(srcm.at[pl.ds(rb, 1)], sw.at[0])
    pltpu.sync_copy(srcm.at[pl.ds(rb + 1, 1)], sw.at[1])
    pltpu.async_copy(table.at[sw.at[0, 0]], rows0, g0)
    pltpu.async_copy(table.at[sw.at[1, 0]], rows1, g1)
    z0.wait()
    z1.wait()
    plsc.subcore_barrier()

    def step(i, carry):
      k = 2 * i
      pltpu.make_async_copy(table.at[sw.at[0, 0]], rows0, g0).wait()

      @pl.when(k + 2 < RW)
      def _():
        pltpu.async_copy(srcm.at[pl.ds(rb + k + 2, 1)], sw.at[0], is0)

      pltpu.sync_copy(ones_v, deg_sh.at[dst_v.at[k]], add=True)
      pltpu.sync_copy(rows0, acc_sh.at[dst_v.at[k]], add=True)

      @pl.when(k + 2 < RW)
      def _():
        pltpu.make_async_copy(srcm.at[pl.ds(rb + k + 2, 1)], sw.at[0], is0).wait()
        pltpu.async_copy(table.at[sw.at[0, 0]], rows0, g0)

      pltpu.make_async_copy(table.at[sw.at[1, 0]], rows1, g1).wait()

      @pl.when(k + 3 < RW)
      def _():
        pltpu.async_copy(srcm.at[pl.ds(rb + k + 3, 1)], sw.at[1], is1)

      pltpu.sync_copy(ones_v, deg_sh.at[dst_v.at[k + 1]], add=True)
      pltpu.sync_copy(rows1, acc_sh.at[dst_v.at[k + 1]], add=True)

      @pl.when(k + 3 < RW)
      def _():
        pltpu.make_async_copy(srcm.at[pl.ds(rb + k + 3, 1)], sw.at[1], is1).wait()
        pltpu.async_copy(table.at[sw.at[1, 0]], rows1, g1)

      return carry

    lax.fori_loop(0, RW // 2, step, 0)
    plsc.subcore_barrier()
    o0 = pltpu.async_copy(acc_sh.at[pl.ds(r0, RPT)], acc_out.at[c, pl.ds(r0, RPT)], zsem)
    o1 = pltpu.async_copy(deg_sh.at[pl.ds(r0, RPT)], deg_out.at[c, pl.ds(r0, RPT)], zsem)
    o0.wait()
    o1.wait()

  f = pl.kernel(
      body,
      out_type=(jax.ShapeDtypeStruct((NC, NP, D_IN), jnp.float32),
                jax.ShapeDtypeStruct((NC, NP), jnp.float32)),
      mesh=_sc_mesh(),
      scratch_types=(
          pltpu.VMEM((2, 1, CHUNK), jnp.int32),
          pltpu.VMEM((RW, CHUNK), jnp.int32),
          pltpu.VMEM((CHUNK, D_IN), jnp.float32),
          pltpu.VMEM((CHUNK, D_IN), jnp.float32),
          pltpu.VMEM((CHUNK,), jnp.float32),
          pltpu.VMEM_SHARED((NP, D_IN), jnp.float32),
          pltpu.VMEM_SHARED((NP,), jnp.float32),
          pltpu.SemaphoreType.DMA,
          pltpu.SemaphoreType.DMA,
          pltpu.SemaphoreType.DMA,
          pltpu.SemaphoreType.DMA,
          pltpu.SemaphoreType.DMA,
      ),
      name="sage_scatter_l0",
  )
  return f(table, srcm, dstm, zfeat, zdeg)


def _scatter_l1(table, srcm, dstm, zp):
  def body(table, srcm, dstm, zp, acc_out, sw, dst_v, rows0, rows1,
           acc_sh, g0, g1, is0, is1, zsem):
    c = lax.axis_index("c")
    s = lax.axis_index("s")
    w = c * NS + s
    r0 = s * RPT
    z0 = pltpu.async_copy(zp.at[pl.ds(r0, RPT)], acc_sh.at[pl.ds(r0, RPT)], zsem)
    rb = w * RW
    pltpu.sync_copy(dstm.at[pl.ds(rb, RW)], dst_v)
    pltpu.sync_copy(srcm.at[pl.ds(rb, 1)], sw.at[0])
    pltpu.sync_copy(srcm.at[pl.ds(rb + 1, 1)], sw.at[1])
    pltpu.async_copy(table.at[sw.at[0, 0]], rows0, g0)
    pltpu.async_copy(table.at[sw.at[1, 0]], rows1, g1)
    z0.wait()
    plsc.subcore_barrier()

    def step(i, carry):
      k = 2 * i
      pltpu.make_async_copy(table.at[sw.at[0, 0]], rows0, g0).wait()

      @pl.when(k + 2 < RW)
      def _():
        pltpu.async_copy(srcm.at[pl.ds(rb + k + 2, 1)], sw.at[0], is0)

      pltpu.sync_copy(rows0, acc_sh.at[dst_v.at[k]], add=True)

      @pl.when(k + 2 < RW)
      def _():
        pltpu.make_async_copy(srcm.at[pl.ds(rb + k + 2, 1)], sw.at[0], is0).wait()
        pltpu.async_copy(table.at[sw.at[0, 0]], rows0, g0)

      pltpu.make_async_copy(table.at[sw.at[1, 0]], rows1, g1).wait()

      @pl.when(k + 3 < RW)
      def _():
        pltpu.async_copy(srcm.at[pl.ds(rb + k + 3, 1)], sw.at[1], is1)

      pltpu.sync_copy(rows1, acc_sh.at[dst_v.at[k + 1]], add=True)

      @pl.when(k + 3 < RW)
      def _():
        pltpu.make_async_copy(srcm.at[pl.ds(rb + k + 3, 1)], sw.at[1], is1).wait()
        pltpu.async_copy(table.at[sw.at[1, 0]], rows1, g1)

      return carry

    lax.fori_loop(0, RW // 2, step, 0)
    plsc.subcore_barrier()
    pltpu.sync_copy(acc_sh.at[pl.ds(r0, RPT)], acc_out.at[c, pl.ds(r0, RPT)])

  f = pl.kernel(
      body,
      out_type=jax.ShapeDtypeStruct((NC, NP, D_HID), jnp.float32),
      mesh=_sc_mesh(),
      scratch_types=(
          pltpu.VMEM((2, 1, CHUNK), jnp.int32),
          pltpu.VMEM((RW, CHUNK), jnp.int32),
          pltpu.VMEM((CHUNK, D_HID), jnp.float32),
          pltpu.VMEM((CHUNK, D_HID), jnp.float32),
          pltpu.VMEM_SHARED((NP, D_HID), jnp.float32),
          pltpu.SemaphoreType.DMA,
          pltpu.SemaphoreType.DMA,
          pltpu.SemaphoreType.DMA,
          pltpu.SemaphoreType.DMA,
          pltpu.SemaphoreType.DMA,
      ),
      name="sage_scatter_l1",
  )
  return f(table, srcm, dstm, zp)


def _dense_body(x_ref, acc_ref, degt_ref, ws0_ref, wn0_ref, bs0_ref, bn0_ref,
                ws1_ref, bs1_ref, bn1_ref, h_ref, s_ref):
  deg = degt_ref[:, 0:1] + degt_ref[:, 1:2]
  inv = 1.0 / jnp.maximum(deg, 1.0)
  hn = (acc_ref[0] + acc_ref[1]) * inv
  t = (jnp.dot(x_ref[...], ws0_ref[...], preferred_element_type=jnp.float32)
       + jnp.dot(hn, wn0_ref[...], preferred_element_type=jnp.float32)
       + bs0_ref[...] + bn0_ref[...])
  h = jnp.maximum(t, 0.0)
  nrm = jnp.sqrt(jnp.sum(h * h, axis=1, keepdims=True))
  h = h / jnp.maximum(nrm, 1e-12)
  h_ref[...] = h
  s_ref[...] = (jnp.dot(h, ws1_ref[...], preferred_element_type=jnp.float32)
                + bs1_ref[...] + bn1_ref[...])


def _dense(x, acc, degt, ws0, wn0, bs0, bn0, ws1, bs1, bn1):
  grid = (N // RB,)
  return pl.pallas_call(
      _dense_body,
      grid=grid,
      in_specs=[
          pl.BlockSpec((RB, D_IN), lambda i: (i, 0)),
          pl.BlockSpec((NC, RB, D_IN), lambda i: (0, i, 0)),
          pl.BlockSpec((RB, NC), lambda i: (i, 0)),
          pl.BlockSpec((D_IN, D_HID), lambda i: (0, 0)),
          pl.BlockSpec((D_IN, D_HID), lambda i: (0, 0)),
          pl.BlockSpec((1, D_HID), lambda i: (0, 0)),
          pl.BlockSpec((1, D_HID), lambda i: (0, 0)),
          pl.BlockSpec((D_HID, NCLS), lambda i: (0, 0)),
          pl.BlockSpec((1, NCLS), lambda i: (0, 0)),
          pl.BlockSpec((1, NCLS), lambda i: (0, 0)),
      ],
      out_specs=[
          pl.BlockSpec((RB, D_HID), lambda i: (i, 0)),
          pl.BlockSpec((RB, NCLS), lambda i: (i, 0)),
      ],
      out_shape=[
          jax.ShapeDtypeStruct((N, D_HID), jnp.float32),
          jax.ShapeDtypeStruct((N, NCLS), jnp.float32),
      ],
      name="sage_dense",
  )(x, acc, degt, ws0, wn0, bs0, bn0, ws1, bs1, bn1)


def _combine_body(s_ref, acc1_ref, degt_ref, wn1_ref, o_ref):
  deg = degt_ref[:, 0:1] + degt_ref[:, 1:2]
  inv = 1.0 / jnp.maximum(deg, 1.0)
  hn = (acc1_ref[0] + acc1_ref[1]) * inv
  o_ref[...] = s_ref[...] + jnp.dot(hn, wn1_ref[...],
                                    preferred_element_type=jnp.float32)


def _combine(s, acc1, degt, wn1):
  grid = (N // RB,)
  return pl.pallas_call(
      _combine_body,
      grid=grid,
      in_specs=[
          pl.BlockSpec((RB, NCLS), lambda i: (i, 0)),
          pl.BlockSpec((NC, RB, D_HID), lambda i: (0, i, 0)),
          pl.BlockSpec((RB, NC), lambda i: (i, 0)),
          pl.BlockSpec((D_HID, NCLS), lambda i: (0, 0)),
      ],
      out_specs=pl.BlockSpec((RB, NCLS), lambda i: (i, 0)),
      out_shape=jax.ShapeDtypeStruct((N, NCLS), jnp.float32),
      name="sage_combine",
  )(s, acc1, degt, wn1)


def kernel(features, edge_index, W_self0, W_neigh0, b_self0, b_neigh0,
           W_self1, W_neigh1, b_self1, b_neigh1):
  src = edge_index[0]
  dst = edge_index[1]
  pad = EP - E
  ar = jnp.arange(pad, dtype=jnp.int32)
  pad_src = (ar * 97) % N
  pad_dst = N + (ar % (NP - N))
  srcm = jnp.concatenate([src, pad_src]).reshape(EP // CHUNK, CHUNK)
  dstm = jnp.concatenate([dst, pad_dst]).reshape(EP // CHUNK, CHUNK)

  zfeat = jnp.zeros((NP, D_IN), jnp.float32)
  zdeg = jnp.zeros((NP,), jnp.float32)

  acc0, deg = _scatter_l0(features, srcm, dstm, zfeat, zdeg)
  degt = deg.T
  h, s = _dense(features, acc0, degt,
                W_self0, W_neigh0,
                b_self0.reshape(1, D_HID), b_neigh0.reshape(1, D_HID),
                W_self1,
                b_self1.reshape(1, NCLS), b_neigh1.reshape(1, NCLS))
  acc1 = _scatter_l1(h, srcm, dstm, zfeat)
  return _combine(s, acc1, degt, W_neigh1)
